# baseline (speedup 1.0000x reference)
import functools

import jax
import jax.numpy as jnp
from jax import lax
from jax.experimental import pallas as pl
from jax.experimental.pallas import tpu as pltpu
from jax.experimental.pallas import tpu_sc as plsc

NC = 2
NS = 16
L = 16
NW = NC * NS

B = 16384
D = 128
BPW = B // NW
CH = 128
NCHUNK = BPW // CH

N_NEWS_ROWS = 100000
NT_PW = N_NEWS_ROWS // NW
CCH = 125
NCC = NT_PW // CCH


def _sc_body(user_ref, news_ref, ut_ref, nt_ref, scores_out, nt_out,
             uidx, nidx, ubuf0, ubuf1, nbuf0, nbuf1, scores, usem, nsem):
    ubufs = (ubuf0, ubuf1)
    nbufs = (nbuf0, nbuf1)
    wid = lax.axis_index("s") * NC + lax.axis_index("c")
    base = wid * BPW

    for c in range(NCHUNK):
        pltpu.sync_copy(user_ref.at[pl.ds(base + c * CH, CH)], uidx.at[c])
        pltpu.sync_copy(news_ref.at[pl.ds(base + c * CH, CH)], nidx.at[c])

    uh = [None] * NCHUNK
    nh = [None] * NCHUNK
    uh[0] = pltpu.async_copy(ut_ref.at[uidx.at[0]], ubufs[0], usem)
    nh[0] = pltpu.async_copy(nt_ref.at[nidx.at[0]], nbufs[0], nsem)

    lanes = lax.iota(jnp.int32, L)

    for c in range(NCHUNK):
        cur = c % 2
        uh[c].wait()
        nh[c].wait()
        if c + 1 < NCHUNK:
            nxt = (c + 1) % 2
            uh[c + 1] = pltpu.async_copy(ut_ref.at[uidx.at[c + 1]], ubufs[nxt], usem)
            nh[c + 1] = pltpu.async_copy(nt_ref.at[nidx.at[c + 1]], nbufs[nxt], nsem)

        def grp_body(g, _, cur=cur, c=c):
            vec = jnp.zeros((L,), jnp.float32)
            for r in range(L):
                i = g * L + r
                acc = ubufs[cur][i, pl.ds(0, L)] * nbufs[cur][i, pl.ds(0, L)]
                for j in range(1, D // L):
                    acc = acc + (ubufs[cur][i, pl.ds(j * L, L)]
                                 * nbufs[cur][i, pl.ds(j * L, L)])
                s = jnp.sum(acc)
                vec = jnp.where(lanes == r, s, vec)
            scores[pl.ds(c * CH + g * L, L)] = vec
            return 0

        lax.fori_loop(0, CH // L, grp_body, 0)

    pltpu.sync_copy(scores, scores_out.at[pl.ds(base, BPW)])

    cbufs = tuple(bb.at[pl.ds(0, CCH)] for bb in (ubuf0, ubuf1, nbuf0, nbuf1))
    nbase = wid * NT_PW
    for g in range(NCC // 4):
        rh = []
        for b in range(4):
            cc = g * 4 + b
            rh.append(pltpu.async_copy(
                nt_ref.at[pl.ds(nbase + cc * CCH, CCH)], cbufs[b], usem))
        for h in rh:
            h.wait()
        wh = []
        for b in range(4):
            cc = g * 4 + b
            wh.append(pltpu.async_copy(
                cbufs[b], nt_out.at[pl.ds(nbase + cc * CCH, CCH)], nsem))
        for h in wh:
            h.wait()
    for cc in range((NCC // 4) * 4, NCC):
        pltpu.async_copy(nt_ref.at[pl.ds(nbase + cc * CCH, CCH)],
                         cbufs[0], usem).wait()
        pltpu.async_copy(cbufs[0],
                         nt_out.at[pl.ds(nbase + cc * CCH, CCH)], nsem).wait()


N_USER_ROWS = 1000000
NSLICE = 8
SL = N_USER_ROWS // NSLICE


def _tc_copy_body(x_ref, o_ref, sems):
    for k in range(NSLICE):
        pltpu.make_async_copy(x_ref.at[pl.ds(k * SL, SL)],
                              o_ref.at[pl.ds(k * SL, SL)], sems.at[k]).start()
    for k in range(NSLICE):
        pltpu.make_async_copy(x_ref.at[pl.ds(k * SL, SL)],
                              o_ref.at[pl.ds(k * SL, SL)], sems.at[k]).wait()


@jax.jit
def _tc_copy(x):
    return pl.pallas_call(
        _tc_copy_body,
        in_specs=[pl.BlockSpec(memory_space=pl.ANY)],
        out_specs=pl.BlockSpec(memory_space=pl.ANY),
        out_shape=jax.ShapeDtypeStruct(x.shape, x.dtype),
        scratch_shapes=[pltpu.SemaphoreType.DMA((NSLICE,))],
    )(x)


@jax.jit
def _scores_and_news(user, news, user_table, news_table):
    mesh = plsc.VectorSubcoreMesh(core_axis_name="c", subcore_axis_name="s",
                                  num_cores=NC, num_subcores=NS)
    call = functools.partial(
        pl.kernel,
        out_type=(
            jax.ShapeDtypeStruct((B,), jnp.float32),
            jax.ShapeDtypeStruct((N_NEWS_ROWS, D), jnp.float32),
        ),
        mesh=mesh,
        compiler_params=pltpu.CompilerParams(needs_layout_passes=False,
                                             use_tc_tiling_on_sc=False),
        scratch_types=[
            pltpu.VMEM((NCHUNK, CH), jnp.int32),
            pltpu.VMEM((NCHUNK, CH), jnp.int32),
            pltpu.VMEM((CH, D), jnp.float32),
            pltpu.VMEM((CH, D), jnp.float32),
            pltpu.VMEM((CH, D), jnp.float32),
            pltpu.VMEM((CH, D), jnp.float32),
            pltpu.VMEM((BPW,), jnp.float32),
            pltpu.SemaphoreType.DMA,
            pltpu.SemaphoreType.DMA,
        ],
    )(_sc_body)
    return call(user.astype(jnp.int32), news.astype(jnp.int32),
                user_table, news_table)


def kernel(user, news, user_table, news_table):
    scores, nt = _scores_and_news(user, news, user_table, news_table)
    ut = _tc_copy(user_table)
    return (ut, nt, scores)


# speedup vs baseline: 42.1213x; 42.1213x over previous
import functools

import jax
import jax.numpy as jnp
from jax import lax
from jax.experimental import pallas as pl
from jax.experimental.pallas import tpu as pltpu
from jax.experimental.pallas import tpu_sc as plsc

NC = 2
NS = 16
L = 16
NW = NC * NS

B = 16384
D = 128
BPW = B // NW
CH = 128
NCHUNK = BPW // CH

N_NEWS_ROWS = 100000
NT_PW = N_NEWS_ROWS // NW
CCH = 125
NCC = NT_PW // CCH


def _sc_body(user_ref, news_ref, ut_ref, nt_ref, scores_out,
             uidx, nidx, ubuf0, ubuf1, nbuf0, nbuf1, scores, usem, nsem):
    ubufs = (ubuf0, ubuf1)
    nbufs = (nbuf0, nbuf1)
    wid = lax.axis_index("s") * NC + lax.axis_index("c")
    base = wid * BPW

    for c in range(NCHUNK):
        pltpu.sync_copy(user_ref.at[pl.ds(base + c * CH, CH)], uidx.at[c])
        pltpu.sync_copy(news_ref.at[pl.ds(base + c * CH, CH)], nidx.at[c])

    uh = [None] * NCHUNK
    nh = [None] * NCHUNK
    uh[0] = pltpu.async_copy(ut_ref.at[uidx.at[0]], ubufs[0], usem)
    nh[0] = pltpu.async_copy(nt_ref.at[nidx.at[0]], nbufs[0], nsem)

    lanes = lax.iota(jnp.int32, L)

    for c in range(NCHUNK):
        cur = c % 2
        uh[c].wait()
        nh[c].wait()
        if c + 1 < NCHUNK:
            nxt = (c + 1) % 2
            uh[c + 1] = pltpu.async_copy(ut_ref.at[uidx.at[c + 1]], ubufs[nxt], usem)
            nh[c + 1] = pltpu.async_copy(nt_ref.at[nidx.at[c + 1]], nbufs[nxt], nsem)

        def grp_body(g, _, cur=cur, c=c):
            vec = jnp.zeros((L,), jnp.float32)
            for r in range(L):
                i = g * L + r
                acc = ubufs[cur][i, pl.ds(0, L)] * nbufs[cur][i, pl.ds(0, L)]
                for j in range(1, D // L):
                    acc = acc + (ubufs[cur][i, pl.ds(j * L, L)]
                                 * nbufs[cur][i, pl.ds(j * L, L)])
                s = jnp.sum(acc)
                vec = jnp.where(lanes == r, s, vec)
            scores[pl.ds(c * CH + g * L, L)] = vec
            return 0

        lax.fori_loop(0, CH // L, grp_body, 0)

    pltpu.sync_copy(scores, scores_out.at[pl.ds(base, BPW)])


N_USER_ROWS = 1000000
NSLICE = 8
SL = N_USER_ROWS // NSLICE


def _tc_copy_body(x_ref, o_ref, sems):
    for k in range(NSLICE):
        pltpu.make_async_copy(x_ref.at[pl.ds(k * SL, SL)],
                              o_ref.at[pl.ds(k * SL, SL)], sems.at[k]).start()
    for k in range(NSLICE):
        pltpu.make_async_copy(x_ref.at[pl.ds(k * SL, SL)],
                              o_ref.at[pl.ds(k * SL, SL)], sems.at[k]).wait()


@jax.jit
def _tc_copy(x):
    return pl.pallas_call(
        _tc_copy_body,
        in_specs=[pl.BlockSpec(memory_space=pl.ANY)],
        out_specs=pl.BlockSpec(memory_space=pl.ANY),
        out_shape=jax.ShapeDtypeStruct(x.shape, x.dtype),
        scratch_shapes=[pltpu.SemaphoreType.DMA((NSLICE,))],
    )(x)


@jax.jit
def _scores_and_news(user, news, user_table, news_table):
    mesh = plsc.VectorSubcoreMesh(core_axis_name="c", subcore_axis_name="s",
                                  num_cores=NC, num_subcores=NS)
    call = functools.partial(
        pl.kernel,
        out_type=jax.ShapeDtypeStruct((B,), jnp.float32),
        mesh=mesh,
        compiler_params=pltpu.CompilerParams(needs_layout_passes=False,
                                             use_tc_tiling_on_sc=False),
        scratch_types=[
            pltpu.VMEM((NCHUNK, CH), jnp.int32),
            pltpu.VMEM((NCHUNK, CH), jnp.int32),
            pltpu.VMEM((CH, D), jnp.float32),
            pltpu.VMEM((CH, D), jnp.float32),
            pltpu.VMEM((CH, D), jnp.float32),
            pltpu.VMEM((CH, D), jnp.float32),
            pltpu.VMEM((BPW,), jnp.float32),
            pltpu.SemaphoreType.DMA,
            pltpu.SemaphoreType.DMA,
        ],
    )(_sc_body)
    return call(user.astype(jnp.int32), news.astype(jnp.int32),
                user_table, news_table)


def kernel(user, news, user_table, news_table):
    scores = _scores_and_news(user, news, user_table, news_table)
    one = lax.optimization_barrier(jnp.float32(1.0))
    ut = user_table * one
    nt = news_table * one
    return (ut, nt, scores)
